# baseline (speedup 1.0000x reference)
import functools

import jax
import jax.numpy as jnp
from jax.experimental import pallas as pl
from jax.experimental.pallas import tpu as pltpu


def _se_kernel(x_ref, w1t_ref, b1_ref, w2t_ref, b2_ref, o_ref, *, inv_hw):
    pooled = jnp.sum(x_ref[...], axis=-1, dtype=jnp.float32) * inv_hw

    h = jnp.dot(pooled, w1t_ref[...], preferred_element_type=jnp.float32)
    h = jnp.maximum(h + b1_ref[...], 0.0)
    s = jnp.dot(h, w2t_ref[...], preferred_element_type=jnp.float32)
    gate = jax.nn.sigmoid(s + b2_ref[...])

    o_ref[...] = (x_ref[...] * gate[:, :, None]).astype(o_ref.dtype)


def _pick_nb(N, per_batch_bytes, budget_bytes):
    cap = max(1, budget_bytes // per_batch_bytes)
    best = 1
    for nb in range(1, N + 1):
        if N % nb or nb > cap:
            continue
        if N // nb < 4 and N >= 4:
            continue
        best = nb
    return best


def kernel(x, w1, b1, w2, b2):
    N, C, H, W = x.shape
    Cr = w1.shape[0]
    HW = H * W

    x3 = x.reshape(N, C, HW)
    w1t = jnp.transpose(w1.reshape(Cr, C))
    w2t = jnp.transpose(w2.reshape(C, Cr))
    b1r = b1.reshape(1, Cr)
    b2r = b2.reshape(1, C)

    hw_pad = ((HW + 127) // 128) * 128
    per_batch = 4 * C * hw_pad * jnp.dtype(x.dtype).itemsize
    nb = _pick_nb(N, per_batch, 24 << 20)
    grid = (N // nb,)

    block = (nb, C, HW)
    x_spec = pl.BlockSpec(block, lambda n: (n, 0, 0))
    w_specs = [
        pl.BlockSpec((C, Cr), lambda n: (0, 0)),
        pl.BlockSpec((1, Cr), lambda n: (0, 0)),
        pl.BlockSpec((Cr, C), lambda n: (0, 0)),
        pl.BlockSpec((1, C), lambda n: (0, 0)),
    ]

    out_flat = pl.pallas_call(
        functools.partial(_se_kernel, inv_hw=1.0 / HW),
        out_shape=jax.ShapeDtypeStruct((N, C, HW), x.dtype),
        grid_spec=pl.GridSpec(
            grid=grid,
            in_specs=[x_spec] + w_specs,
            out_specs=x_spec,
        ),
        compiler_params=pltpu.CompilerParams(
            dimension_semantics=("parallel",),
            vmem_limit_bytes=min(nb * per_batch + (8 << 20), 48 << 20),
        ),
    )(x3, w1t, b1r, w2t, b2r)

    return out_flat.reshape(N, C, H, W)
